# baseline (speedup 1.0000x reference)
import functools

import jax
import jax.numpy as jnp
from jax import lax
from jax.experimental import pallas as pl
from jax.experimental.pallas import tpu as pltpu
from jax.experimental.pallas import tpu_sc as plsc

N, D, S, H, O, B = 100000, 128, 16, 128, 128, 1024
F = B + B * S
NW = 32
BW = B // NW
RW = F // NW
CH = 16
NCH = RW // CH
HZ = CH * S // 2


def _mm_body(x_ref, w_ref, o_ref):
    o_ref[...] = jnp.dot(x_ref[...], w_ref[...],
                         preferred_element_type=jnp.float32)


def _dense_z(x, w1b):
    bm = 2000
    return pl.pallas_call(
        _mm_body,
        grid=(N // bm,),
        in_specs=[
            pl.BlockSpec((bm, D), lambda i: (i, 0)),
            pl.BlockSpec((D, H), lambda i: (0, 0)),
        ],
        out_specs=pl.BlockSpec((bm, H), lambda i: (i, 0)),
        out_shape=jax.ShapeDtypeStruct((N, H), jnp.float32),
    )(x, w1b)


NQ = 5
NZ = F * S // NW // 128
ZRING = 4


def _sc_gather_max(batch32, neigh32, x, z):
    mesh = plsc.VectorSubcoreMesh(core_axis_name="c", subcore_axis_name="s")

    @functools.partial(
        pl.kernel,
        out_type=[
            jax.ShapeDtypeStruct((F, D), jnp.float32),
            jax.ShapeDtypeStruct((F, H), jnp.float32),
        ],
        mesh=mesh,
        compiler_params=pltpu.CompilerParams(use_tc_tiling_on_sc=False),
        scratch_types=[
            pltpu.VMEM((BW,), jnp.int32),
            pltpu.VMEM((BW, S), jnp.int32),
            pltpu.VMEM((NQ, 128), jnp.int32),
            pltpu.VMEM((NQ * 128, S), jnp.int32),
            pltpu.VMEM((NZ, 128), jnp.int32),
            pltpu.VMEM((128, D), jnp.float32),
            pltpu.VMEM((128, D), jnp.float32),
            pltpu.VMEM((128, H), jnp.float32),
            pltpu.VMEM((128, H), jnp.float32),
            pltpu.VMEM((128, H), jnp.float32),
            pltpu.VMEM((128, H), jnp.float32),
            pltpu.VMEM((CH, H), jnp.float32),
            pltpu.VMEM((CH, H), jnp.float32),
            pltpu.SemaphoreType.DMA,
            pltpu.SemaphoreType.DMA,
            pltpu.SemaphoreType.DMA,
            pltpu.SemaphoreType.DMA,
            pltpu.SemaphoreType.DMA,
            pltpu.SemaphoreType.DMA,
            pltpu.SemaphoreType.DMA,
            pltpu.SemaphoreType.DMA,
            pltpu.SemaphoreType.DMA,
            pltpu.SemaphoreType.DMA,
            pltpu.SemaphoreType.DMA,
        ],
    )
    def k(batch_hbm, neigh_hbm, x_hbm, z_hbm, xf_hbm, mf_hbm,
          bs_v, nb_v, fs_v, nf_v, zi_v, xr0_v, xr1_v,
          zr0_v, zr1_v, zr2_v, zr3_v, mc0_v, mc1_v,
          sng, sxg0, sxg1, sxw0, sxw1, szg0, szg1, szg2, szg3, smw0, smw1):
        w = lax.axis_index("c") * 16 + lax.axis_index("s")
        xr = [xr0_v, xr1_v]
        zr = [zr0_v, zr1_v, zr2_v, zr3_v]
        mc = [mc0_v, mc1_v]
        sxg, sxw = [sxg0, sxg1], [sxw0, sxw1]
        szg, smw = [szg0, szg1, szg2, szg3], [smw0, smw1]
        nbase = B + w * (BW * S)

        pltpu.sync_copy(batch_hbm.at[pl.ds(w * BW, BW)], bs_v)
        pltpu.async_copy(neigh_hbm.at[bs_v], nb_v, sng).wait()

        fs_v[0, pl.ds(0, 16)] = bs_v[pl.ds(0, 16)]
        fs_v[0, pl.ds(16, 16)] = bs_v[pl.ds(16, 16)]
        for q in range(2, 8):
            fs_v[NQ - 1, pl.ds(q * 16, 16)] = jnp.zeros((16,), jnp.int32)

        def flat_nb(r, carry):
            e = BW + r * S
            fs_v[e // 128, pl.ds(e % 128, 16)] = nb_v[r, :]
            return carry
        lax.fori_loop(0, BW, flat_nb, 0)

        nf_cps = [
            pltpu.async_copy(neigh_hbm.at[fs_v.at[q]],
                             nf_v.at[pl.ds(q * 128, 128)], sng)
            for q in range(NQ)
        ]
        xg_cps = [pltpu.async_copy(x_hbm.at[fs_v.at[q]], xr[q % 2], sxg[q % 2])
                  for q in range(2)]

        for cp in nf_cps:
            cp.wait()

        def flat_nf(r, carry):
            zi_v[r // 8, pl.ds((r % 8) * S, S)] = nf_v[r, :]
            return carry
        lax.fori_loop(0, RW, flat_nf, 0)

        for b in range(ZRING):
            pltpu.async_copy(z_hbm.at[zi_v.at[b]], zr[b], szg[b])

        xg_cps[0].wait()
        pltpu.async_copy(xr[0].at[pl.ds(0, BW)],
                         xf_hbm.at[pl.ds(w * BW, BW)], sxw0)
        pltpu.async_copy(xr[0].at[pl.ds(BW, 96)],
                         xf_hbm.at[pl.ds(nbase, 96)], sxw0)
        xg_cps[1].wait()
        pltpu.async_copy(xr[1], xf_hbm.at[pl.ds(nbase + 96, 128)], sxw1)
        for q in range(2, NQ):
            s = q % 2
            pltpu.make_async_copy(x_hbm.at[pl.ds(0, 128)], xr[s], sxw[s]).wait()
            pltpu.async_copy(x_hbm.at[fs_v.at[q]], xr[s], sxg[s]).wait()
            rows = 128 if q < NQ - 1 else BW
            pltpu.async_copy(xr[s].at[pl.ds(0, rows)],
                             xf_hbm.at[pl.ds(nbase + q * 128 - BW, rows)],
                             sxw[s])

        def seg_chunk(zc, mcb, half):
            def seg(i, carry):
                accs0 = tuple(zc[i * S, pl.ds(h * 16, 16)] for h in range(8))

                def red(r, accs):
                    return tuple(
                        jnp.maximum(a, zc[i * S + r, pl.ds(h * 16, 16)])
                        for h, a in enumerate(accs))
                accs = lax.fori_loop(1, S, red, accs0)
                for h in range(8):
                    mcb[half * 8 + i, pl.ds(h * 16, 16)] = accs[h]
                return carry
            lax.fori_loop(0, 8, seg, 0)

        def zloop(t, carry):
            for b in range(ZRING):
                j = 4 * t + b
                m = b // 2
                pltpu.make_async_copy(z_hbm.at[pl.ds(0, 128)], zr[b],
                                      szg[b]).wait()
                if b % 2 == 0:
                    @pl.when(t > 0)
                    def _():
                        pltpu.make_async_copy(mf_hbm.at[pl.ds(0, CH)],
                                              mc[m], smw[m]).wait()
                seg_chunk(zr[b], mc[m], b % 2)
                if b % 2 == 1:
                    c = 2 * t + m
                    row0 = jnp.where(c < 2, w * BW + c * CH,
                                     nbase + (c - 2) * CH)
                    pltpu.async_copy(mc[m], mf_hbm.at[pl.ds(row0, CH)], smw[m])

                @pl.when(t < NZ // ZRING - 1)
                def _():
                    pltpu.async_copy(z_hbm.at[zi_v.at[j + ZRING]],
                                     zr[b], szg[b])
            return carry
        lax.fori_loop(0, NZ // ZRING, zloop, 0)

        pltpu.make_async_copy(mf_hbm.at[pl.ds(0, CH)], mc0_v, smw0).wait()
        pltpu.make_async_copy(mf_hbm.at[pl.ds(0, CH)], mc1_v, smw1).wait()
        pltpu.make_async_copy(x_hbm.at[pl.ds(0, BW)],
                              xr0_v.at[pl.ds(0, BW)], sxw0).wait()
        pltpu.make_async_copy(x_hbm.at[pl.ds(0, 128)], xr1_v, sxw1).wait()

    return k(batch32, neigh32, x, z)


def _tail_body(xfb_ref, xfn_ref, mfb_ref, mfn_ref, w1a_ref, b1_ref,
               w2a_ref, w2b_ref, b2_ref, wo_ref, bo_ref, o_ref):
    h1b = jnp.maximum(
        jnp.dot(xfb_ref[...], w1a_ref[...], preferred_element_type=jnp.float32)
        + b1_ref[...] + mfb_ref[...], 0.0)
    h1n = jnp.maximum(
        jnp.dot(xfn_ref[...], w1a_ref[...], preferred_element_type=jnp.float32)
        + b1_ref[...] + mfn_ref[...], 0.0)
    u = jnp.dot(h1b, w2a_ref[...],
                preferred_element_type=jnp.float32) + b2_ref[...]
    wn = jnp.dot(h1n, w2b_ref[...], preferred_element_type=jnp.float32)
    m2 = jnp.max(wn.reshape(-1, S, H), axis=1)
    o_ref[...] = jnp.dot(u + m2, wo_ref[...],
                         preferred_element_type=jnp.float32) + bo_ref[...]


def _dense_tail(xf, mf, w1a, b1, w2a, w2b, b2, wo, bo):
    bm = 64
    full = lambda i: (0, 0)
    return pl.pallas_call(
        _tail_body,
        grid=(B // bm,),
        in_specs=[
            pl.BlockSpec((bm, D), lambda i: (i, 0)),
            pl.BlockSpec((bm * S, D), lambda i: (i + 1, 0)),
            pl.BlockSpec((bm, H), lambda i: (i, 0)),
            pl.BlockSpec((bm * S, H), lambda i: (i + 1, 0)),
            pl.BlockSpec((D, H), full),
            pl.BlockSpec((1, H), full),
            pl.BlockSpec((H, H), full),
            pl.BlockSpec((H, H), full),
            pl.BlockSpec((1, H), full),
            pl.BlockSpec((H, O), full),
            pl.BlockSpec((1, O), full),
        ],
        out_specs=pl.BlockSpec((bm, O), lambda i: (i, 0)),
        out_shape=jax.ShapeDtypeStruct((B, O), jnp.float32),
    )(xf, xf, mf, mf, w1a, b1.reshape(1, H), w2a, w2b, b2.reshape(1, H),
      wo, bo.reshape(1, O))


@jax.jit
def kernel(x, neigh, batch, W1, b1, W2, b2, Wo, bo):
    neigh32 = neigh.astype(jnp.int32)
    batch32 = batch.astype(jnp.int32)
    w1a, w1b = W1[:D], W1[D:]
    w2a, w2b = W2[:H], W2[H:]

    z = _dense_z(x, w1b)
    xf, mf = _sc_gather_max(batch32, neigh32, x, z)
    return _dense_tail(xf, mf, w1a, b1, w2a, w2b, b2, Wo, bo)
